# Initial kernel scaffold; baseline (speedup 1.0000x reference)
import jax
import jax.numpy as jnp
from jax.experimental import pallas as pl


def kernel(density, cube_shape, grid_xyz, sphere_mask):
    raise NotImplementedError("write your pallas kernel here")


# speedup vs baseline: 8.2951x; 8.2951x over previous
import functools

import jax
import jax.numpy as jnp
from jax import lax
from jax.experimental import pallas as pl
from jax.experimental.pallas import tpu as pltpu
from jax.experimental.pallas import tpu_sc as plsc

NXYZ = 48
G = NXYZ * NXYZ * NXYZ
CELLS = 128
NC, NS, L = 2, 16, 16
NW = NC * NS
CPW = CELLS // NW
RAD = 4
NEG = -1e9
THRESH = -1e8
NSTEP = G // L
NCH = 16
CH_STEP = NSTEP // NCH
CH = CH_STEP * L
DPAD = 16
BIG = 1 << 30


def _body(dens, maskf, grid, out, vbuf, mb0, mb1, idxv, xyzv, outbuf,
          semd, semm0, semm1, semg):
    cid = lax.axis_index("c")
    sid = lax.axis_index("s")
    w = sid * NC + cid
    _worker(w, dens, maskf, grid, out, vbuf, mb0, mb1, idxv, xyzv, outbuf,
            semd, semm0, semm1, semg)


def _worker(w, dens, maskf, grid, out, vbuf, mb0, mb1, idxv, xyzv, outbuf,
            semd, semm0, semm1, semg):
    lane = lax.iota(jnp.int32, L)
    neg = jnp.float32(NEG)
    thresh = jnp.float32(THRESH)
    big = jnp.int32(BIG)
    gidxs, scores, alives = [], [], []

    for t in range(CPW):
        cell = w * CPW + t
        cpd = pltpu.async_copy(dens.at[pl.ds(cell * G, G)], vbuf, semd)
        mcopies = [None, None]
        mcopies[0] = pltpu.async_copy(maskf.at[pl.ds(0, CH)], mb0, semm0)
        cpd.wait()

        curmax = jnp.full((L,), neg, jnp.float32)
        curidx = jnp.zeros((L,), jnp.int32)
        for ch in range(NCH):
            mcopies[ch % 2].wait()
            if ch + 1 < NCH:
                mb_next = mb1 if (ch + 1) % 2 else mb0
                sem_next = semm1 if (ch + 1) % 2 else semm0
                mcopies[(ch + 1) % 2] = pltpu.async_copy(
                    maskf.at[pl.ds((ch + 1) * CH, CH)], mb_next, sem_next)
            mb = mb1 if ch % 2 else mb0

            def step(u, carry, _ch=ch, _mb=mb):
                cm, ci = carry
                off = _ch * CH + u * L
                v = vbuf[pl.ds(off, L)]
                m = _mb[pl.ds(u * L, L)]
                v = jnp.where(m > jnp.float32(0.5), v, neg)
                vbuf[pl.ds(off, L)] = v
                gt = v > cm
                cm = jnp.where(gt, v, cm)
                ci = jnp.where(gt, lane + off, ci)
                return cm, ci

            curmax, curidx = lax.fori_loop(0, CH_STEP, step, (curmax, curidx))

        gmax = jnp.max(curmax)
        gidx = jnp.min(jnp.where(curmax == gmax, curidx, big))
        alive1 = gmax >= thresh

        i0 = gidx // (NXYZ * NXYZ)
        rem = gidx - i0 * (NXYZ * NXYZ)
        j0 = rem // NXYZ
        k0 = rem - j0 * NXYZ
        ks = jnp.clip(k0 - RAD, 0, NXYZ - L)
        kwin = ks + lane
        kmask = jnp.abs(kwin - k0) <= RAD

        def supp(s, carry):
            di = s // (2 * RAD + 1) - RAD
            dj = s % (2 * RAD + 1) - RAD
            ii = i0 + di
            jj = j0 + dj
            valid = (ii >= 0) & (ii < NXYZ) & (jj >= 0) & (jj < NXYZ)
            iic = jnp.clip(ii, 0, NXYZ - 1)
            jjc = jnp.clip(jj, 0, NXYZ - 1)
            base = (iic * NXYZ + jjc) * NXYZ + ks
            vv = vbuf[pl.ds(base, L)]
            vbuf[pl.ds(base, L)] = jnp.where(kmask & valid, neg, vv)
            return carry

        lax.fori_loop(0, (2 * RAD + 1) * (2 * RAD + 1), supp, 0)

        def step2(u, carry):
            cm, ci = carry
            off = u * L
            v = vbuf[pl.ds(off, L)]
            gt = v > cm
            cm = jnp.where(gt, v, cm)
            ci = jnp.where(gt, lane + off, ci)
            return cm, ci

        curmax2 = jnp.full((L,), neg, jnp.float32)
        curidx2 = jnp.zeros((L,), jnp.int32)
        curmax2, curidx2 = lax.fori_loop(0, NSTEP, step2, (curmax2, curidx2))
        gmax2 = jnp.max(curmax2)
        gidx2 = jnp.min(jnp.where(curmax2 == gmax2, curidx2, big))
        alive2 = alive1 & (gmax2 >= thresh)

        gidxs.append(jnp.where(alive1, gidx, 0))
        gidxs.append(jnp.where(alive2, gidx2, 0))
        scores.append(jnp.where(alive1, gmax, neg))
        scores.append(jnp.where(alive2, gmax2, neg))
        alives.append(alive1)
        alives.append(alive2)

    idxacc = jnp.zeros((L,), jnp.int32)
    for p in range(2 * CPW):
        idxacc = jnp.where(lane == p, gidxs[p], idxacc)
    idxv[...] = idxacc
    pltpu.async_copy(grid.at[idxv], xyzv, semg).wait()

    afl = [jnp.where(a, jnp.float32(1.0), jnp.float32(0.0)) for a in alives]
    v0 = jnp.zeros((L,), jnp.float32)
    v1 = jnp.zeros((L,), jnp.float32)
    v2 = jnp.zeros((L,), jnp.float32)
    for p in range(2 * CPW):
        row = xyzv[p, :]
        for q in range(3):
            s = 3 * p + q
            val = row[q] * afl[p]
            if s < 16:
                v0 = jnp.where(lane == s, val, v0)
            else:
                v1 = jnp.where(lane == (s - 16), val, v1)
    for p in range(2 * CPW):
        v1 = jnp.where(lane == (8 + p), scores[p], v1)
        v2 = jnp.where(lane == p, afl[p], v2)
    outbuf[pl.ds(0, L)] = v0
    outbuf[pl.ds(16, L)] = v1
    outbuf[pl.ds(32, L)] = v2
    pltpu.sync_copy(outbuf, out.at[w])


@functools.lru_cache(maxsize=None)
def _sc_call():
    return pl.kernel(
        _body,
        out_type=jax.ShapeDtypeStruct((NW, 48), jnp.float32),
        mesh=plsc.VectorSubcoreMesh(core_axis_name="c", subcore_axis_name="s",
                                    num_cores=NC, num_subcores=NS),
        compiler_params=pltpu.CompilerParams(needs_layout_passes=False, use_tc_tiling_on_sc=False),
        scratch_types=[
            pltpu.VMEM((G,), jnp.float32),
            pltpu.VMEM((CH,), jnp.float32),
            pltpu.VMEM((CH,), jnp.float32),
            pltpu.VMEM((L,), jnp.int32),
            pltpu.VMEM((L, DPAD), jnp.float32),
            pltpu.VMEM((48,), jnp.float32),
            pltpu.SemaphoreType.DMA,
            pltpu.SemaphoreType.DMA,
            pltpu.SemaphoreType.DMA,
            pltpu.SemaphoreType.DMA,
        ],
    )


def kernel(density, cube_shape, grid_xyz, sphere_mask):
    del cube_shape
    B, N, C, _ = density.shape
    dens = density.reshape(CELLS * G)
    maskf = sphere_mask.astype(jnp.float32)
    grid16 = jnp.pad(grid_xyz, ((0, 0), (0, DPAD - 3)))
    out = _sc_call()(dens, maskf, grid16)
    peaks_xyz = out[:, :24].reshape(B, N, C, 2, 3)
    peaks_score = out[:, 24:32].reshape(B, N, C, 2)
    peaks_mask = out[:, 32:40].reshape(B, N, C, 2) > 0.5
    return peaks_xyz, peaks_score, peaks_mask


# speedup vs baseline: 12.9009x; 1.5552x over previous
import functools

import jax
import jax.numpy as jnp
from jax import lax
from jax.experimental import pallas as pl
from jax.experimental.pallas import tpu as pltpu
from jax.experimental.pallas import tpu_sc as plsc

NXYZ = 48
G = NXYZ * NXYZ * NXYZ
CELLS = 128
NC, NS, L = 2, 16, 16
NW = NC * NS
CPW = CELLS // NW
RAD = 4
NEG = -1e9
THRESH = -1e8
NSTEP = G // L
NCH = 16
CH_STEP = NSTEP // NCH
CH = CH_STEP * L
DPAD = 16
BIG = 1 << 30
UNROLL = 4


def _merge(cms, cis):
    cm, ci = cms[0], cis[0]
    for g in range(1, len(cms)):
        m = jnp.maximum(cm, cms[g])
        both = (cm == m) & (cms[g] == m)
        pick_b = (cms[g] == m) & ~(cm == m)
        ni = jnp.where(both, jnp.minimum(ci, cis[g]),
                       jnp.where(pick_b, cis[g], ci))
        cm, ci = m, ni
    return cm, ci


def _body(dens, maskf, grid, out, vbuf, mb0, mb1, idxv, xyzv, outbuf,
          semd, semm0, semm1, semg):
    cid = lax.axis_index("c")
    sid = lax.axis_index("s")
    w = sid * NC + cid
    _worker(w, dens, maskf, grid, out, vbuf, mb0, mb1, idxv, xyzv, outbuf,
            semd, semm0, semm1, semg)


def _worker(w, dens, maskf, grid, out, vbuf, mb0, mb1, idxv, xyzv, outbuf,
            semd, semm0, semm1, semg):
    lane = lax.iota(jnp.int32, L)
    neg = jnp.float32(NEG)
    thresh = jnp.float32(THRESH)
    big = jnp.int32(BIG)
    gidxs, scores, alives = [], [], []

    for t in range(CPW):
        cell = w * CPW + t
        cpd = pltpu.async_copy(dens.at[pl.ds(cell * G, G)], vbuf, semd)
        mcopies = [None, None]
        mcopies[0] = pltpu.async_copy(maskf.at[pl.ds(0, CH)], mb0, semm0)
        cpd.wait()

        curmax = tuple(jnp.full((L,), neg, jnp.float32) for _ in range(UNROLL))
        curidx = tuple(jnp.zeros((L,), jnp.int32) for _ in range(UNROLL))
        for ch in range(NCH):
            mcopies[ch % 2].wait()
            if ch + 1 < NCH:
                mb_next = mb1 if (ch + 1) % 2 else mb0
                sem_next = semm1 if (ch + 1) % 2 else semm0
                mcopies[(ch + 1) % 2] = pltpu.async_copy(
                    maskf.at[pl.ds((ch + 1) * CH, CH)], mb_next, sem_next)
            mb = mb1 if ch % 2 else mb0

            def step(u, carry, _ch=ch, _mb=mb):
                cms, cis = carry
                ncms, ncis = [], []
                for g in range(UNROLL):
                    off = _ch * CH + (u * UNROLL + g) * L
                    moff = (u * UNROLL + g) * L
                    v = vbuf[pl.ds(off, L)]
                    m = _mb[pl.ds(moff, L)]
                    v = jnp.where(m > jnp.float32(0.5), v, neg)
                    vbuf[pl.ds(off, L)] = v
                    gt = v > cms[g]
                    ncms.append(jnp.where(gt, v, cms[g]))
                    ncis.append(jnp.where(gt, lane + off, cis[g]))
                return tuple(ncms), tuple(ncis)

            curmax, curidx = lax.fori_loop(
                0, CH_STEP // UNROLL, step, (curmax, curidx))

        cm1, ci1 = _merge(curmax, curidx)
        gmax = jnp.max(cm1)
        gidx = jnp.min(jnp.where(cm1 == gmax, ci1, big))
        alive1 = gmax >= thresh

        i0 = gidx // (NXYZ * NXYZ)
        rem = gidx - i0 * (NXYZ * NXYZ)
        j0 = rem // NXYZ
        k0 = rem - j0 * NXYZ
        ks = jnp.clip(k0 - RAD, 0, NXYZ - L)
        kwin = ks + lane
        kmask = jnp.abs(kwin - k0) <= RAD

        def supp(s, carry):
            di = s // (2 * RAD + 1) - RAD
            dj = s % (2 * RAD + 1) - RAD
            ii = i0 + di
            jj = j0 + dj
            valid = (ii >= 0) & (ii < NXYZ) & (jj >= 0) & (jj < NXYZ)
            iic = jnp.clip(ii, 0, NXYZ - 1)
            jjc = jnp.clip(jj, 0, NXYZ - 1)
            base = (iic * NXYZ + jjc) * NXYZ + ks
            vv = vbuf[pl.ds(base, L)]
            vbuf[pl.ds(base, L)] = jnp.where(kmask & valid, neg, vv)
            return carry

        lax.fori_loop(0, (2 * RAD + 1) * (2 * RAD + 1), supp, 0)

        def step2(u, carry):
            cms, cis = carry
            ncms, ncis = [], []
            for g in range(UNROLL):
                off = (u * UNROLL + g) * L
                v = vbuf[pl.ds(off, L)]
                gt = v > cms[g]
                ncms.append(jnp.where(gt, v, cms[g]))
                ncis.append(jnp.where(gt, lane + off, cis[g]))
            return tuple(ncms), tuple(ncis)

        curmax2 = tuple(jnp.full((L,), neg, jnp.float32) for _ in range(UNROLL))
        curidx2 = tuple(jnp.zeros((L,), jnp.int32) for _ in range(UNROLL))
        curmax2, curidx2 = lax.fori_loop(0, NSTEP // UNROLL, step2,
                                         (curmax2, curidx2))
        cm2, ci2 = _merge(curmax2, curidx2)
        gmax2 = jnp.max(cm2)
        gidx2 = jnp.min(jnp.where(cm2 == gmax2, ci2, big))
        alive2 = alive1 & (gmax2 >= thresh)

        gidxs.append(jnp.where(alive1, gidx, 0))
        gidxs.append(jnp.where(alive2, gidx2, 0))
        scores.append(jnp.where(alive1, gmax, neg))
        scores.append(jnp.where(alive2, gmax2, neg))
        alives.append(alive1)
        alives.append(alive2)

    idxacc = jnp.zeros((L,), jnp.int32)
    for p in range(2 * CPW):
        idxacc = jnp.where(lane == p, gidxs[p], idxacc)
    idxv[...] = idxacc
    pltpu.async_copy(grid.at[idxv], xyzv, semg).wait()

    afl = [jnp.where(a, jnp.float32(1.0), jnp.float32(0.0)) for a in alives]
    v0 = jnp.zeros((L,), jnp.float32)
    v1 = jnp.zeros((L,), jnp.float32)
    v2 = jnp.zeros((L,), jnp.float32)
    for p in range(2 * CPW):
        row = xyzv[p, :]
        for q in range(3):
            s = 3 * p + q
            val = row[q] * afl[p]
            if s < 16:
                v0 = jnp.where(lane == s, val, v0)
            else:
                v1 = jnp.where(lane == (s - 16), val, v1)
    for p in range(2 * CPW):
        v1 = jnp.where(lane == (8 + p), scores[p], v1)
        v2 = jnp.where(lane == p, afl[p], v2)
    outbuf[pl.ds(0, L)] = v0
    outbuf[pl.ds(16, L)] = v1
    outbuf[pl.ds(32, L)] = v2
    pltpu.sync_copy(outbuf, out.at[w])


@functools.lru_cache(maxsize=None)
def _sc_call():
    return pl.kernel(
        _body,
        out_type=jax.ShapeDtypeStruct((NW, 48), jnp.float32),
        mesh=plsc.VectorSubcoreMesh(core_axis_name="c", subcore_axis_name="s",
                                    num_cores=NC, num_subcores=NS),
        compiler_params=pltpu.CompilerParams(needs_layout_passes=False, use_tc_tiling_on_sc=False),
        scratch_types=[
            pltpu.VMEM((G,), jnp.float32),
            pltpu.VMEM((CH,), jnp.float32),
            pltpu.VMEM((CH,), jnp.float32),
            pltpu.VMEM((L,), jnp.int32),
            pltpu.VMEM((L, DPAD), jnp.float32),
            pltpu.VMEM((48,), jnp.float32),
            pltpu.SemaphoreType.DMA,
            pltpu.SemaphoreType.DMA,
            pltpu.SemaphoreType.DMA,
            pltpu.SemaphoreType.DMA,
        ],
    )


def kernel(density, cube_shape, grid_xyz, sphere_mask):
    del cube_shape
    B, N, C, _ = density.shape
    dens = density.reshape(CELLS * G)
    maskf = sphere_mask.astype(jnp.float32)
    grid16 = jnp.pad(grid_xyz, ((0, 0), (0, DPAD - 3)))
    out = _sc_call()(dens, maskf, grid16)
    peaks_xyz = out[:, :24].reshape(B, N, C, 2, 3)
    peaks_score = out[:, 24:32].reshape(B, N, C, 2)
    peaks_mask = out[:, 32:40].reshape(B, N, C, 2) > 0.5
    return peaks_xyz, peaks_score, peaks_mask
